# Initial kernel scaffold; baseline (speedup 1.0000x reference)
import jax
import jax.numpy as jnp
from jax.experimental import pallas as pl


def kernel(x, edge_index, edge_attr, batch, W, b, We, gamma, beta, run_mean, run_var):
    raise NotImplementedError("write your pallas kernel here")


# speedup vs baseline: 3.1870x; 3.1870x over previous
import functools

import jax
import jax.numpy as jnp
from jax import lax
from jax.experimental import pallas as pl
from jax.experimental.pallas import tpu as pltpu
from jax.experimental.pallas import tpu_sc as plsc

N = 10000
E = 320000
D = 128
DE = 4

NC = 2
NS = 16
NW = NC * NS
EPW = E // NW
CHUNK = 80
NCHUNK = EPW // CHUNK
NPAD = 10240
ROWS_PT = NPAD // NS
ZROWS = 128


def _ef_body(ea_ref, we_ref, out_ref):
    out_ref[...] = jnp.dot(ea_ref[...], we_ref[...],
                           preferred_element_type=jnp.float32)


def _edge_feat(edge_attr, We):
    B = 4000
    return pl.pallas_call(
        _ef_body,
        grid=(E // B,),
        in_specs=[pl.BlockSpec((B, DE), lambda i: (i, 0)),
                  pl.BlockSpec((DE, D), lambda i: (0, 0))],
        out_specs=pl.BlockSpec((B, D), lambda i: (i, 0)),
        out_shape=jax.ShapeDtypeStruct((E, D), jnp.float32),
    )(edge_attr, We)


def _hmm_body(x_ref, w_ref, b_ref, out_ref):
    out_ref[...] = jnp.dot(x_ref[...], w_ref[...],
                           preferred_element_type=jnp.float32) + b_ref[...]


def _hmm(x, W, b2):
    B = 2000
    return pl.pallas_call(
        _hmm_body,
        grid=(N // B,),
        in_specs=[pl.BlockSpec((B, D), lambda i: (i, 0)),
                  pl.BlockSpec((D, D), lambda i: (0, 0)),
                  pl.BlockSpec((1, D), lambda i: (0, 0))],
        out_specs=pl.BlockSpec((B, D), lambda i: (i, 0)),
        out_shape=jax.ShapeDtypeStruct((N, D), jnp.float32),
    )(x, W, b2)


def _upd_common(x_ref, h_ref, a0_ref, a1_ref, g_ref, be_ref, rm_ref, rv_ref):
    u = h_ref[...] + a0_ref[...] + a1_ref[...]
    scale = g_ref[...] * lax.rsqrt(rv_ref[...] + 1e-5)
    u = (u - rm_ref[...]) * scale + be_ref[...]
    u = jnp.maximum(u, 0.0)
    return 0.5 * x_ref[...] + 0.5 * u


def _updmm_body(x_ref, h_ref, a0_ref, a1_ref, g_ref, be_ref, rm_ref, rv_ref,
                w_ref, b_ref, xo_ref, ho_ref):
    xn = _upd_common(x_ref, h_ref, a0_ref, a1_ref, g_ref, be_ref, rm_ref, rv_ref)
    xo_ref[...] = xn
    ho_ref[...] = jnp.dot(xn, w_ref[...],
                          preferred_element_type=jnp.float32) + b_ref[...]


def _upd_last_body(x_ref, h_ref, a0_ref, a1_ref, g_ref, be_ref, rm_ref, rv_ref,
                   xo_ref):
    xo_ref[...] = _upd_common(x_ref, h_ref, a0_ref, a1_ref,
                              g_ref, be_ref, rm_ref, rv_ref)


def _update_mm(x, h, a0, a1, g2, be2, rm2, rv2, W, b2):
    B = 2000
    row = lambda i: (i, 0)
    fixed = lambda i: (0, 0)
    return pl.pallas_call(
        _updmm_body,
        grid=(N // B,),
        in_specs=[pl.BlockSpec((B, D), row), pl.BlockSpec((B, D), row),
                  pl.BlockSpec((B, D), row), pl.BlockSpec((B, D), row),
                  pl.BlockSpec((1, D), fixed), pl.BlockSpec((1, D), fixed),
                  pl.BlockSpec((1, D), fixed), pl.BlockSpec((1, D), fixed),
                  pl.BlockSpec((D, D), fixed), pl.BlockSpec((1, D), fixed)],
        out_specs=[pl.BlockSpec((B, D), row), pl.BlockSpec((B, D), row)],
        out_shape=[jax.ShapeDtypeStruct((N, D), jnp.float32),
                   jax.ShapeDtypeStruct((N, D), jnp.float32)],
    )(x, h, a0, a1, g2, be2, rm2, rv2, W, b2)


def _update_last(x, h, a0, a1, g2, be2, rm2, rv2):
    B = 2000
    row = lambda i: (i, 0)
    fixed = lambda i: (0, 0)
    return pl.pallas_call(
        _upd_last_body,
        grid=(N // B,),
        in_specs=[pl.BlockSpec((B, D), row), pl.BlockSpec((B, D), row),
                  pl.BlockSpec((B, D), row), pl.BlockSpec((B, D), row),
                  pl.BlockSpec((1, D), fixed), pl.BlockSpec((1, D), fixed),
                  pl.BlockSpec((1, D), fixed), pl.BlockSpec((1, D), fixed)],
        out_specs=pl.BlockSpec((B, D), row),
        out_shape=jax.ShapeDtypeStruct((N, D), jnp.float32),
    )(x, h, a0, a1, g2, be2, rm2, rv2)


def _edge_pass_body(h_hbm, src_hbm, dst_hbm, e_hbm, out_hbm,
                    srcv, dstv, hrows, erows, zbuf, agg_sh, sem_g, sem_e):
    c = lax.axis_index("c")
    s = lax.axis_index("s")
    wid = s * NC + c

    def zrow(j, _):
        for t in range(D // 16):
            zbuf[j, pl.ds(t * 16, 16)] = jnp.zeros((16,), jnp.float32)
        return 0
    lax.fori_loop(0, ZROWS, zrow, 0)
    for k in range(ROWS_PT // ZROWS):
        pltpu.sync_copy(zbuf, agg_sh.at[pl.ds(s * ROWS_PT + k * ZROWS, ZROWS)])
    plsc.subcore_barrier()

    def chunk(i, _):
        base = wid * EPW + i * CHUNK
        pltpu.sync_copy(src_hbm.at[pl.ds(base, CHUNK)], srcv)
        pltpu.sync_copy(dst_hbm.at[pl.ds(base, CHUNK)], dstv)
        cg = pltpu.async_copy(h_hbm.at[srcv], hrows, sem_g)
        ce = pltpu.async_copy(e_hbm.at[pl.ds(base, CHUNK)], erows, sem_e)
        cg.wait()
        ce.wait()

        def row(j, _):
            for t in range(D // 16):
                sl = pl.ds(t * 16, 16)
                hrows[j, sl] = jnp.maximum(hrows[j, sl] + erows[j, sl], 0.0)
            return 0
        lax.fori_loop(0, CHUNK, row, 0)
        pltpu.sync_copy(hrows, agg_sh.at[dstv], add=True)
        return 0
    lax.fori_loop(0, NCHUNK, chunk, 0)
    plsc.subcore_barrier()

    pltpu.sync_copy(agg_sh.at[pl.ds(s * ROWS_PT, ROWS_PT)],
                    out_hbm.at[c, pl.ds(s * ROWS_PT, ROWS_PT)])


_edge_pass = functools.partial(
    pl.kernel,
    out_type=jax.ShapeDtypeStruct((NC, NPAD, D), jnp.float32),
    mesh=plsc.VectorSubcoreMesh(core_axis_name="c", subcore_axis_name="s"),
    scratch_types=[
        pltpu.VMEM((CHUNK,), jnp.int32),
        pltpu.VMEM((CHUNK,), jnp.int32),
        pltpu.VMEM((CHUNK, D), jnp.float32),
        pltpu.VMEM((CHUNK, D), jnp.float32),
        pltpu.VMEM((ZROWS, D), jnp.float32),
        pltpu.VMEM_SHARED((NPAD, D), jnp.float32),
        pltpu.SemaphoreType.DMA,
        pltpu.SemaphoreType.DMA,
    ],
)(_edge_pass_body)


def kernel(x, edge_index, edge_attr, batch, W, b, We, gamma, beta,
           run_mean, run_var):
    src = edge_index[0]
    dst = edge_index[1]
    b2 = b.reshape(1, D)
    g2 = gamma.reshape(1, D)
    be2 = beta.reshape(1, D)
    rm2 = run_mean.reshape(1, D)
    rv2 = run_var.reshape(1, D)

    e = _edge_feat(edge_attr, We)
    h = _hmm(x, W, b2)
    for i in range(4):
        aggs = _edge_pass(h, src, dst, e)
        a0 = aggs[0, :N]
        a1 = aggs[1, :N]
        if i < 3:
            x, h = _update_mm(x, h, a0, a1, g2, be2, rm2, rv2, W, b2)
        else:
            x = _update_last(x, h, a0, a1, g2, be2, rm2, rv2)
    return x
